# Initial kernel scaffold; baseline (speedup 1.0000x reference)
import jax
import jax.numpy as jnp
from jax.experimental import pallas as pl


def kernel(X, La, U, ve, W_w, W_b, bn_gamma, bn_beta, MLP_w, MLP_b):
    raise NotImplementedError("write your pallas kernel here")


# speedup vs baseline: 2.0986x; 2.0986x over previous
import functools

import jax
import jax.numpy as jnp
from jax import lax
from jax.experimental import pallas as pl
from jax.experimental.pallas import tpu as pltpu


def _fused_kernel(ve_ref, la_ref, u_ref, x_ref,
                  w1_ref, b1_ref, gamma_ref, beta_ref,
                  w2_ref, b2_ref,
                  out_ref, hid_ref):
    f32 = jnp.float32
    U = u_ref[...]
    X = x_ref[...]

    T2 = lax.dot_general(X, U, (((0,), (0,)), ((), ())),
                         preferred_element_type=f32)

    vla = jnp.power(la_ref[...], ve_ref[0])
    Tv2 = T2 * vla

    Tw2 = lax.dot_general(w1_ref[...], Tv2, (((1,), (0,)), ((), ())),
                          preferred_element_type=f32)

    H = lax.dot_general(U, Tw2, (((1,), (1,)), ((), ())),
                        preferred_element_type=f32) + b1_ref[...]
    hid_ref[...] = H

    mean = jnp.mean(H, axis=0, keepdims=True)
    var = jnp.mean(jnp.square(H - mean), axis=0, keepdims=True)
    Hn = (H - mean) * lax.rsqrt(var + 1e-5)
    Hn = Hn * gamma_ref[...] + beta_ref[...]

    Hr = jnp.maximum(Hn, 0.0)

    logits = lax.dot_general(Hr, w2_ref[...], (((1,), (1,)), ((), ())),
                             preferred_element_type=f32) + b2_ref[...]

    m = jnp.max(logits, axis=1, keepdims=True)
    z = logits - m
    lse = jnp.log(jnp.sum(jnp.exp(z), axis=1, keepdims=True))
    out_ref[...] = z - lse


def kernel(X, La, U, ve, W_w, W_b, bn_gamma, bn_beta, MLP_w, MLP_b):
    N, in_c = X.shape
    hidden = W_w.shape[0]
    out_c = MLP_w.shape[0]

    vmem = pl.BlockSpec(memory_space=pltpu.MemorySpace.VMEM)
    smem = pl.BlockSpec(memory_space=pltpu.MemorySpace.SMEM)

    out, hidden_emd = pl.pallas_call(
        _fused_kernel,
        out_shape=(
            jax.ShapeDtypeStruct((N, out_c), jnp.float32),
            jax.ShapeDtypeStruct((N, hidden), jnp.float32),
        ),
        in_specs=[smem] + [vmem] * 9,
        out_specs=(vmem, vmem),
    )(
        ve.astype(jnp.float32).reshape(1),
        La.reshape(1, N).astype(jnp.float32),
        U.astype(jnp.float32),
        X.astype(jnp.float32),
        W_w.astype(jnp.float32),
        W_b.reshape(1, hidden).astype(jnp.float32),
        bn_gamma.reshape(1, hidden).astype(jnp.float32),
        bn_beta.reshape(1, hidden).astype(jnp.float32),
        MLP_w.astype(jnp.float32),
        MLP_b.reshape(1, out_c).astype(jnp.float32),
    )
    return out, hidden_emd
